# baseline (speedup 1.0000x reference)
import functools

import jax
import jax.numpy as jnp
from jax import lax
from jax.experimental import pallas as pl
from jax.experimental.pallas import tpu as pltpu, tpu_sc as plsc

_PANEL = 128
_L = 16


def _make_panel_gather(V, D, B):
    info = plsc.get_sparse_core_info()
    NC, NS = info.num_cores, info.num_subcores
    NW = NC * NS
    n_full = V // _PANEL
    tail_start = n_full * _PANEL
    tail_n = V - tail_start
    tail_owner = n_full % NW
    mesh = plsc.VectorSubcoreMesh(core_axis_name="c", subcore_axis_name="s")

    @functools.partial(
        pl.kernel,
        mesh=mesh,
        out_type=jax.ShapeDtypeStruct((B, D), jnp.float32),
        compiler_params=pltpu.CompilerParams(needs_layout_passes=False),
        scratch_types=[
            pltpu.VMEM((B,), jnp.int32),
            pltpu.VMEM((B + _L,), jnp.int32),
            pltpu.VMEM((B + _L,), jnp.int32),
            pltpu.VMEM((B + _L,), jnp.int32),
            pltpu.VMEM((B + _L,), jnp.int32),
            pltpu.VMEM((D, _PANEL), jnp.float32),
            pltpu.VMEM((D, _PANEL), jnp.float32),
            pltpu.VMEM((tail_n, D), jnp.float32),
            pltpu.VMEM((_L, D), jnp.float32),
            pltpu.SemaphoreType.DMA,
            pltpu.SemaphoreType.DMA,
        ],
    )
    def gather_k(
        table_t_hbm, idx_hbm, tail_hbm, out_hbm,
        idx_all_v, myidx_v, mypos_v, subc_v, subj_v,
        panel_a, panel_b, tail_v, stage_v, psem, osem,
    ):
        w = lax.axis_index("s") * NC + lax.axis_index("c")
        pltpu.sync_copy(idx_hbm, idx_all_v)
        pltpu.sync_copy(tail_hbm, tail_v)
        iota = lax.iota(jnp.int32, _L)
        dvecs = [iota + k * _L for k in range(D // _L)]

        def compress_append(ref_a, ref_b, base, xa, xb, m):
            mi = m.astype(jnp.int32)
            inc = plsc.cumsum(mi)
            pos = (inc - mi) + base
            plsc.store_scatter(ref_a, [pos], xa, mask=m)
            plsc.store_scatter(ref_b, [pos], xb, mask=m)
            return base + inc[_L - 1]

        @pl.loop(0, B // _L, init_carry=0)
        def cnt(k, cnt):
            v = idx_all_v[pl.ds(k * _L, _L)]
            mine = (lax.shift_right_logical(v, 7) & (NW - 1)) == w
            return compress_append(
                myidx_v, mypos_v, cnt, v, iota + k * _L, mine
            )

        n_trips = lax.div(cnt + (_L - 1), _L)
        n_slots = lax.div((n_full - 1) - w, NW) + 1

        def issue_panel(p, buf):
            col = pl.multiple_of(p * _PANEL, _PANEL)
            pltpu.async_copy(
                table_t_hbm.at[:, pl.ds(col, _PANEL)], buf, psem
            )

        def wait_panel():
            pltpu.make_async_copy(
                table_t_hbm.at[:, pl.ds(0, _PANEL)], panel_a, psem
            ).wait()

        def drain_block():
            pltpu.make_async_copy(
                stage_v.at[pl.ds(0, _L)], out_hbm.at[pl.ds(0, _L)], osem
            ).wait()

        def build_sublist(p):
            @pl.loop(0, n_trips, init_carry=0)
            def scnt(t, scnt):
                v = myidx_v[pl.ds(t * _L, _L)]
                pos = mypos_v[pl.ds(t * _L, _L)]
                m = lax.shift_right_logical(v, 7) == p
                return compress_append(
                    subc_v, subj_v, scnt, v & (_PANEL - 1), pos, m
                )

            @pl.when(scnt > 0)
            def _():
                c16 = subc_v[pl.ds(0, _L)]
                j16 = subj_v[pl.ds(0, _L)]
                plsc.store_scatter(
                    subc_v, [scnt + iota], jnp.full((_L,), c16[0], jnp.int32)
                )
                plsc.store_scatter(
                    subj_v, [scnt + iota], jnp.full((_L,), j16[0], jnp.int32)
                )

            return scnt

        def extract_blocks(eblocks, cur):
            @pl.loop(0, eblocks)
            def _ex(e, /):
                cv = subc_v[pl.ds(e * _L, _L)]
                jv = subj_v[pl.ds(e * _L, _L)]
                for l in range(_L):
                    cvec = jnp.full((_L,), cv[l], jnp.int32)
                    for k in range(D // _L):
                        stage_v[l, pl.ds(k * _L, _L)] = (
                            plsc.load_gather(cur, [dvecs[k], cvec])
                        )
                    pltpu.async_copy(
                        stage_v.at[pl.ds(l, 1)],
                        out_hbm.at[pl.ds(jv[l], 1)],
                        osem,
                    )
                drain_block()

        issue_panel(w, panel_a)

        @pl.loop(0, n_slots)
        def _slot(i, /):
            p = w + i * NW
            scnt = build_sublist(p)
            eblocks = lax.div(scnt + (_L - 1), _L)

            def process(cur, nxt):
                @pl.when(i + 1 < n_slots)
                def _():
                    issue_panel(p + NW, nxt)

                wait_panel()
                extract_blocks(eblocks, cur)

            @pl.when(i % 2 == 0)
            def _():
                process(panel_a, panel_b)

            @pl.when(i % 2 == 1)
            def _():
                process(panel_b, panel_a)

        @pl.when(w == tail_owner)
        def _():
            scnt = build_sublist(n_full)
            eblocks = lax.div(scnt + (_L - 1), _L)

            @pl.loop(0, eblocks)
            def _ex(e, /):
                cv = subc_v[pl.ds(e * _L, _L)]
                jv = subj_v[pl.ds(e * _L, _L)]
                for l in range(_L):
                    pltpu.async_copy(
                        tail_v.at[pl.ds(cv[l], 1)],
                        out_hbm.at[pl.ds(jv[l], 1)],
                        osem,
                    )
                pltpu.make_async_copy(
                    tail_v.at[pl.ds(0, _L)],
                    out_hbm.at[pl.ds(0, _L)],
                    osem,
                ).wait()

    return gather_k


def kernel(data, channelindex):
    V, D = data.shape
    (B,) = channelindex.shape
    idx = channelindex.astype(jnp.int32)
    tail = data[(V // _PANEL) * _PANEL :, :]
    return _make_panel_gather(V, D, B)(data.T, idx, tail)
